# baseline (speedup 1.0000x reference)
import functools

import jax
import jax.numpy as jnp
from jax import lax
from jax.experimental import pallas as pl
from jax.experimental.pallas import tpu as pltpu
from jax.experimental.pallas import tpu_sc as plsc

_GROUPS = 8
_NUM_VARS = 64
_VAR_DIM = 64
_PROJ_DIM = 32

_BLK = 512
_HALVES = 2

_info = plsc.get_sparse_core_info()
_NC = _info.num_cores
_NS = _info.num_subcores
_NW = _NC * _NS
_CHUNK = 128


def _logits_argmax_body(x_ref, wpre_ref, bpre_ref, wwp_ref, bwp_ref, idx_ref):
    x = x_ref[...]
    h = jnp.dot(x, wpre_ref[...], preferred_element_type=jnp.float32)
    h = h + bpre_ref[...]
    logits = jnp.dot(h, wwp_ref[...], preferred_element_type=jnp.float32)
    logits = logits + bwp_ref[...]
    cols = []
    for g in range(_GROUPS):
        sub = logits[:, g * _NUM_VARS:(g + 1) * _NUM_VARS]
        k = jnp.argmax(sub, axis=-1).astype(jnp.int32) + g * _NUM_VARS
        cols.append(k[:, None])
    idx_ref[...] = jnp.concatenate(cols, axis=-1)


def _postproject_body(q_ref, wpost_ref, bpost_ref, out_ref):
    out = jnp.dot(q_ref[...], wpost_ref[...], preferred_element_type=jnp.float32)
    out_ref[...] = out + bpost_ref[...]


def _make_sc_gather(rows, nch):
    mesh = plsc.VectorSubcoreMesh(core_axis_name="c", subcore_axis_name="s")

    @functools.partial(
        pl.kernel,
        mesh=mesh,
        compiler_params=pltpu.CompilerParams(use_tc_tiling_on_sc=False),
        out_type=jax.ShapeDtypeStruct((rows, _VAR_DIM), jnp.bfloat16),
        scratch_types=[
            pltpu.VMEM((nch, _CHUNK), jnp.int32),
            pltpu.VMEM((_CHUNK, _VAR_DIM), jnp.bfloat16),
            pltpu.VMEM((_CHUNK, _VAR_DIM), jnp.bfloat16),
            pltpu.SemaphoreType.DMA,
            pltpu.SemaphoreType.DMA,
        ],
    )
    def gather(idx_hbm, cb_hbm, out_hbm, idx_v, buf0, buf1, sem0, sem1):
        wid = lax.axis_index("s") * _NC + lax.axis_index("c")
        pltpu.sync_copy(idx_hbm.at[wid], idx_v)
        bufs = (buf0, buf1)
        sems = (sem0, sem1)
        base = wid * nch * _CHUNK
        cps = [None] * nch
        cps[0] = pltpu.async_copy(cb_hbm.at[idx_v.at[0]], bufs[0], sems[0])
        for j in range(nch):
            if j + 1 < nch:
                cps[j + 1] = pltpu.async_copy(
                    cb_hbm.at[idx_v.at[j + 1]], bufs[(j + 1) % 2], sems[(j + 1) % 2])
            cps[j].wait()
            pltpu.sync_copy(bufs[j % 2],
                            out_hbm.at[pl.ds(base + j * _CHUNK, _CHUNK)])

    return gather


def kernel(x, W_pre, b_pre, W_wp, b_wp, codebook, W_post, b_post):
    B, T, IN_DIM = x.shape
    OUT_DIM = W_post.shape[1]
    BT = B * T
    HBT = BT // _HALVES
    xf = x.reshape(BT, IN_DIM)
    cb16 = codebook.astype(jnp.bfloat16)
    wpost16 = W_post.astype(jnp.bfloat16)
    bpre2 = b_pre.reshape(1, -1)
    bwp2 = b_wp.reshape(1, -1)
    bpost2 = b_post.reshape(1, -1)

    rows = HBT * _GROUPS
    nch = rows // (_NW * _CHUNK)
    sc_gather = _make_sc_gather(rows, nch)

    argmax_call = pl.pallas_call(
        _logits_argmax_body,
        grid=(HBT // _BLK,),
        in_specs=[
            pl.BlockSpec((_BLK, IN_DIM), lambda i: (i, 0)),
            pl.BlockSpec((IN_DIM, _PROJ_DIM), lambda i: (0, 0)),
            pl.BlockSpec((1, _PROJ_DIM), lambda i: (0, 0)),
            pl.BlockSpec((_PROJ_DIM, _GROUPS * _NUM_VARS), lambda i: (0, 0)),
            pl.BlockSpec((1, _GROUPS * _NUM_VARS), lambda i: (0, 0)),
        ],
        out_specs=pl.BlockSpec((_BLK, _GROUPS), lambda i: (i, 0)),
        out_shape=jax.ShapeDtypeStruct((HBT, _GROUPS), jnp.int32),
    )

    post_call = pl.pallas_call(
        _postproject_body,
        grid=(HBT // _BLK,),
        in_specs=[
            pl.BlockSpec((_BLK, _GROUPS * _VAR_DIM), lambda i: (i, 0)),
            pl.BlockSpec((_GROUPS * _VAR_DIM, OUT_DIM), lambda i: (0, 0)),
            pl.BlockSpec((1, OUT_DIM), lambda i: (0, 0)),
        ],
        out_specs=pl.BlockSpec((_BLK, OUT_DIM), lambda i: (i, 0)),
        out_shape=jax.ShapeDtypeStruct((HBT, OUT_DIM), jnp.float32),
    )

    idx = [argmax_call(xf[h * HBT:(h + 1) * HBT], W_pre, bpre2, W_wp, bwp2)
           for h in range(_HALVES)]
    q = [sc_gather(idx[h].reshape(_NW, nch, _CHUNK), cb16)
         for h in range(_HALVES)]
    outs = [post_call(q[h].reshape(HBT, _GROUPS * _VAR_DIM), wpost16, bpost2)
            for h in range(_HALVES)]
    return jnp.concatenate(outs, axis=0).reshape(B, T, OUT_DIM)


# speedup vs baseline: 1.5000x; 1.5000x over previous
import functools

import jax
import jax.numpy as jnp
from jax import lax
from jax.experimental import pallas as pl
from jax.experimental.pallas import tpu as pltpu
from jax.experimental.pallas import tpu_sc as plsc

_GROUPS = 8
_NUM_VARS = 64
_VAR_DIM = 64
_PROJ_DIM = 32
_PAIRS = 4
_PAIR_DIM = 2 * _VAR_DIM

_BLK = 512

_info = plsc.get_sparse_core_info()
_NC = _info.num_cores
_NS = _info.num_subcores
_NW = _NC * _NS
_CHUNK = 128


def _pair_table_body(cb_ref, pt_ref):
    cb0 = cb_ref[:_NUM_VARS, :]
    cb1 = cb_ref[_NUM_VARS:, :]
    a = jnp.broadcast_to(cb0[:, None, :], (_NUM_VARS, _NUM_VARS, _VAR_DIM))
    b = jnp.broadcast_to(cb1[None, :, :], (_NUM_VARS, _NUM_VARS, _VAR_DIM))
    a = a.reshape(_NUM_VARS * _NUM_VARS, _VAR_DIM)
    b = b.reshape(_NUM_VARS * _NUM_VARS, _VAR_DIM)
    pt_ref[...] = jnp.concatenate([a, b], axis=-1)


def _logits_argmax_body(x_ref, wpre_ref, bpre_ref, wwp_ref, bwp_ref, idx_ref):
    x = x_ref[...]
    h = jnp.dot(x, wpre_ref[...], preferred_element_type=jnp.float32)
    h = h + bpre_ref[...]
    logits = jnp.dot(h, wwp_ref[...], preferred_element_type=jnp.float32)
    logits = logits + bwp_ref[...]
    ks = []
    for g in range(_GROUPS):
        sub = logits[:, g * _NUM_VARS:(g + 1) * _NUM_VARS]
        ks.append(jnp.argmax(sub, axis=-1).astype(jnp.int32))
    cols = []
    for p in range(_PAIRS):
        pidx = ks[2 * p] * _NUM_VARS + ks[2 * p + 1] + p * _NUM_VARS * _NUM_VARS
        cols.append(pidx[:, None])
    idx_ref[...] = jnp.concatenate(cols, axis=-1)


def _postproject_body(q_ref, wp_ref, bpost_ref, out_ref):
    r = q_ref[...].reshape(_BLK, _PAIRS, _PAIR_DIM)
    acc = None
    for p in range(_PAIRS):
        qp = r[:, p, :].astype(jnp.bfloat16)
        wp = wp_ref[p].astype(jnp.bfloat16)
        d = jnp.dot(qp, wp, preferred_element_type=jnp.float32)
        acc = d if acc is None else acc + d
    out_ref[...] = acc + bpost_ref[...]


def _make_sc_gather(rows, nch):
    mesh = plsc.VectorSubcoreMesh(core_axis_name="c", subcore_axis_name="s")

    @functools.partial(
        pl.kernel,
        mesh=mesh,
        out_type=jax.ShapeDtypeStruct((rows, _PAIR_DIM), jnp.float32),
        scratch_types=[
            pltpu.VMEM((nch, _CHUNK), jnp.int32),
            pltpu.VMEM((_CHUNK, _PAIR_DIM), jnp.float32),
            pltpu.VMEM((_CHUNK, _PAIR_DIM), jnp.float32),
            pltpu.SemaphoreType.DMA,
            pltpu.SemaphoreType.DMA,
        ],
    )
    def gather(idx_hbm, pt_hbm, out_hbm, idx_v, buf0, buf1, sem0, sem1):
        wid = lax.axis_index("s") * _NC + lax.axis_index("c")
        pltpu.sync_copy(idx_hbm.at[wid], idx_v)
        bufs = (buf0, buf1)
        sems = (sem0, sem1)
        base = wid * nch * _CHUNK
        cps = [None] * nch
        cps[0] = pltpu.async_copy(pt_hbm.at[idx_v.at[0]], bufs[0], sems[0])
        for j in range(nch):
            if j + 1 < nch:
                cps[j + 1] = pltpu.async_copy(
                    pt_hbm.at[idx_v.at[j + 1]], bufs[(j + 1) % 2], sems[(j + 1) % 2])
            cps[j].wait()
            pltpu.sync_copy(bufs[j % 2],
                            out_hbm.at[pl.ds(base + j * _CHUNK, _CHUNK)])

    return gather


def kernel(x, W_pre, b_pre, W_wp, b_wp, codebook, W_post, b_post):
    B, T, IN_DIM = x.shape
    OUT_DIM = W_post.shape[1]
    BT = B * T
    xf = x.reshape(BT, IN_DIM)
    bpre2 = b_pre.reshape(1, -1)
    bwp2 = b_wp.reshape(1, -1)
    bpost2 = b_post.reshape(1, -1)
    NN = _NUM_VARS * _NUM_VARS

    pair_table = pl.pallas_call(
        _pair_table_body,
        grid=(_PAIRS,),
        in_specs=[pl.BlockSpec((2 * _NUM_VARS, _VAR_DIM), lambda i: (i, 0))],
        out_specs=pl.BlockSpec((NN, _PAIR_DIM), lambda i: (i, 0)),
        out_shape=jax.ShapeDtypeStruct((_PAIRS * NN, _PAIR_DIM), jnp.float32),
    )(codebook)

    idx = pl.pallas_call(
        _logits_argmax_body,
        grid=(BT // _BLK,),
        in_specs=[
            pl.BlockSpec((_BLK, IN_DIM), lambda i: (i, 0)),
            pl.BlockSpec((IN_DIM, _PROJ_DIM), lambda i: (0, 0)),
            pl.BlockSpec((1, _PROJ_DIM), lambda i: (0, 0)),
            pl.BlockSpec((_PROJ_DIM, _GROUPS * _NUM_VARS), lambda i: (0, 0)),
            pl.BlockSpec((1, _GROUPS * _NUM_VARS), lambda i: (0, 0)),
        ],
        out_specs=pl.BlockSpec((_BLK, _PAIRS), lambda i: (i, 0)),
        out_shape=jax.ShapeDtypeStruct((BT, _PAIRS), jnp.int32),
    )(xf, W_pre, bpre2, W_wp, bwp2)

    rows = BT * _PAIRS
    nch = rows // (_NW * _CHUNK)
    q = _make_sc_gather(rows, nch)(idx.reshape(_NW, nch, _CHUNK), pair_table)

    out = pl.pallas_call(
        _postproject_body,
        grid=(BT // _BLK,),
        in_specs=[
            pl.BlockSpec((_BLK * _PAIRS, _PAIR_DIM), lambda i: (i, 0)),
            pl.BlockSpec((_PAIRS, _PAIR_DIM, OUT_DIM), lambda i: (0, 0, 0)),
            pl.BlockSpec((1, OUT_DIM), lambda i: (0, 0)),
        ],
        out_specs=pl.BlockSpec((_BLK, OUT_DIM), lambda i: (i, 0)),
        out_shape=jax.ShapeDtypeStruct((BT, OUT_DIM), jnp.float32),
    )(q, W_post.reshape(_PAIRS, _PAIR_DIM, OUT_DIM), bpost2)

    return out.reshape(B, T, OUT_DIM)


# speedup vs baseline: 1.9108x; 1.2738x over previous
import functools

import jax
import jax.numpy as jnp
from jax import lax
from jax.experimental import pallas as pl
from jax.experimental.pallas import tpu as pltpu
from jax.experimental.pallas import tpu_sc as plsc

_GROUPS = 8
_NUM_VARS = 64
_VAR_DIM = 64
_PROJ_DIM = 32
_PAIRS = 4
_PAIR_DIM = 2 * _VAR_DIM

_BLK = 512

_info = plsc.get_sparse_core_info()
_NC = _info.num_cores
_NS = _info.num_subcores
_NW = _NC * _NS
_CHUNK = 128


def _pair_table_body(cb_ref, pt_ref):
    cb0 = cb_ref[:_NUM_VARS, :]
    cb1 = cb_ref[_NUM_VARS:, :]
    a = jnp.broadcast_to(cb0[:, None, :], (_NUM_VARS, _NUM_VARS, _VAR_DIM))
    b = jnp.broadcast_to(cb1[None, :, :], (_NUM_VARS, _NUM_VARS, _VAR_DIM))
    a = a.reshape(_NUM_VARS * _NUM_VARS, _VAR_DIM)
    b = b.reshape(_NUM_VARS * _NUM_VARS, _VAR_DIM)
    pt_ref[...] = jnp.concatenate([a, b], axis=-1)


def _logits_argmax_body(x_ref, wpre_ref, bpre_ref, wwp_ref, bwp_ref, idx_ref):
    x = x_ref[...]
    h = jnp.dot(x, wpre_ref[...], preferred_element_type=jnp.float32)
    h = h + bpre_ref[...]
    logits = jnp.dot(h, wwp_ref[...], preferred_element_type=jnp.float32)
    logits = logits + bwp_ref[...]
    rows = x.shape[0]
    iot = jax.lax.broadcasted_iota(jnp.int32, (rows, _NUM_VARS), 1)
    ks = []
    for g in range(_GROUPS):
        sub = logits[:, g * _NUM_VARS:(g + 1) * _NUM_VARS]
        m = jnp.max(sub, axis=-1, keepdims=True)
        ks.append(jnp.min(jnp.where(sub == m, iot, _NUM_VARS), axis=-1))
    parts = []
    for p in range(_PAIRS):
        pidx = ks[2 * p] * _NUM_VARS + ks[2 * p + 1] + p * _NUM_VARS * _NUM_VARS
        parts.append(pidx.reshape(rows // _CHUNK, _CHUNK))
    idx_ref[...] = jnp.concatenate(parts, axis=0)


def _postproject_body(q_ref, wp_ref, bpost_ref, out_ref):
    r = q_ref[...].reshape(_PAIRS, _BLK, _PAIR_DIM)
    acc = None
    for p in range(_PAIRS):
        qp = r[p].astype(jnp.bfloat16)
        wp = wp_ref[p].astype(jnp.bfloat16)
        d = jnp.dot(qp, wp, preferred_element_type=jnp.float32)
        acc = d if acc is None else acc + d
    out_ref[...] = acc + bpost_ref[...]


def _make_sc_gather(rows, nch):
    mesh = plsc.VectorSubcoreMesh(core_axis_name="c", subcore_axis_name="s")

    @functools.partial(
        pl.kernel,
        mesh=mesh,
        out_type=jax.ShapeDtypeStruct((rows, _PAIR_DIM), jnp.float32),
        scratch_types=[
            pltpu.VMEM((nch, _CHUNK), jnp.int32),
            pltpu.VMEM((_CHUNK, _PAIR_DIM), jnp.float32),
            pltpu.VMEM((_CHUNK, _PAIR_DIM), jnp.float32),
            pltpu.SemaphoreType.DMA,
            pltpu.SemaphoreType.DMA,
        ],
    )
    def gather(idx_hbm, pt_hbm, out_hbm, idx_v, buf0, buf1, sem0, sem1):
        wid = lax.axis_index("s") * _NC + lax.axis_index("c")
        pltpu.sync_copy(idx_hbm.at[wid], idx_v)
        bufs = (buf0, buf1)
        sems = (sem0, sem1)
        base = wid * nch * _CHUNK
        cps = [None] * nch
        cps[0] = pltpu.async_copy(pt_hbm.at[idx_v.at[0]], bufs[0], sems[0])
        for j in range(nch):
            if j + 1 < nch:
                cps[j + 1] = pltpu.async_copy(
                    pt_hbm.at[idx_v.at[j + 1]], bufs[(j + 1) % 2], sems[(j + 1) % 2])
            cps[j].wait()
            pltpu.sync_copy(bufs[j % 2],
                            out_hbm.at[pl.ds(base + j * _CHUNK, _CHUNK)])

    return gather


def kernel(x, W_pre, b_pre, W_wp, b_wp, codebook, W_post, b_post):
    B, T, IN_DIM = x.shape
    OUT_DIM = W_post.shape[1]
    BT = B * T
    xf = x.reshape(BT, IN_DIM)
    bpre2 = b_pre.reshape(1, -1)
    bwp2 = b_wp.reshape(1, -1)
    bpost2 = b_post.reshape(1, -1)
    NN = _NUM_VARS * _NUM_VARS

    pair_table = pl.pallas_call(
        _pair_table_body,
        grid=(_PAIRS,),
        in_specs=[pl.BlockSpec((2 * _NUM_VARS, _VAR_DIM), lambda i: (i, 0))],
        out_specs=pl.BlockSpec((NN, _PAIR_DIM), lambda i: (i, 0)),
        out_shape=jax.ShapeDtypeStruct((_PAIRS * NN, _PAIR_DIM), jnp.float32),
    )(codebook)

    idx = pl.pallas_call(
        _logits_argmax_body,
        grid=(BT // _BLK,),
        in_specs=[
            pl.BlockSpec((_BLK, IN_DIM), lambda i: (i, 0)),
            pl.BlockSpec((IN_DIM, _PROJ_DIM), lambda i: (0, 0)),
            pl.BlockSpec((1, _PROJ_DIM), lambda i: (0, 0)),
            pl.BlockSpec((_PROJ_DIM, _GROUPS * _NUM_VARS), lambda i: (0, 0)),
            pl.BlockSpec((1, _GROUPS * _NUM_VARS), lambda i: (0, 0)),
        ],
        out_specs=pl.BlockSpec((_BLK * _PAIRS // _CHUNK, _CHUNK), lambda i: (i, 0)),
        out_shape=jax.ShapeDtypeStruct((BT * _PAIRS // _CHUNK, _CHUNK), jnp.int32),
    )(xf, W_pre, bpre2, W_wp, bwp2)

    rows = BT * _PAIRS
    nch = rows // (_NW * _CHUNK)
    q = _make_sc_gather(rows, nch)(idx.reshape(_NW, nch, _CHUNK), pair_table)

    out = pl.pallas_call(
        _postproject_body,
        grid=(BT // _BLK,),
        in_specs=[
            pl.BlockSpec((_BLK * _PAIRS, _PAIR_DIM), lambda i: (i, 0)),
            pl.BlockSpec((_PAIRS, _PAIR_DIM, OUT_DIM), lambda i: (0, 0, 0)),
            pl.BlockSpec((1, OUT_DIM), lambda i: (0, 0)),
        ],
        out_specs=pl.BlockSpec((_BLK, OUT_DIM), lambda i: (i, 0)),
        out_shape=jax.ShapeDtypeStruct((BT, OUT_DIM), jnp.float32),
    )(q, W_post.reshape(_PAIRS, _PAIR_DIM, OUT_DIM), bpost2)

    return out.reshape(B, T, OUT_DIM)


# speedup vs baseline: 2.0976x; 1.0978x over previous
import functools

import jax
import jax.numpy as jnp
from jax import lax
from jax.experimental import pallas as pl
from jax.experimental.pallas import tpu as pltpu
from jax.experimental.pallas import tpu_sc as plsc

_GROUPS = 8
_NUM_VARS = 64
_VAR_DIM = 64
_PROJ_DIM = 32
_PAIRS = 4
_PAIR_DIM = 2 * _VAR_DIM

_BLK = 512

_info = plsc.get_sparse_core_info()
_NC = _info.num_cores
_NS = _info.num_subcores
_NW = _NC * _NS
_CHUNK = 128


def _pair_table_body(cb_ref, pt_ref):
    cb0 = cb_ref[:_NUM_VARS, :]
    cb1 = cb_ref[_NUM_VARS:, :]
    a = jnp.broadcast_to(cb0[:, None, :], (_NUM_VARS, _NUM_VARS, _VAR_DIM))
    b = jnp.broadcast_to(cb1[None, :, :], (_NUM_VARS, _NUM_VARS, _VAR_DIM))
    a = a.reshape(_NUM_VARS * _NUM_VARS, _VAR_DIM)
    b = b.reshape(_NUM_VARS * _NUM_VARS, _VAR_DIM)
    pt_ref[...] = jnp.concatenate([a, b], axis=-1)


def _logits_argmax_body(x_ref, wpre_ref, bpre_ref, wwp_ref, bwp_ref, idx_ref):
    x = x_ref[...]
    h = jnp.dot(x, wpre_ref[...], preferred_element_type=jnp.float32)
    h = h + bpre_ref[...]
    logits = jnp.dot(h, wwp_ref[...], preferred_element_type=jnp.float32)
    logits = logits + bwp_ref[...]
    rows = x.shape[0]
    iot = jax.lax.broadcasted_iota(jnp.int32, (rows, _NUM_VARS), 1)
    rev = _NUM_VARS - 1 - iot
    minint = jnp.int32(-2147483648)
    ks = []
    for g in range(_GROUPS):
        sub = logits[:, g * _NUM_VARS:(g + 1) * _NUM_VARS]
        s = jax.lax.bitcast_convert_type(sub, jnp.int32)
        key = jnp.where(s >= 0, s, ~(s ^ minint))
        packed = (key & jnp.int32(~63)) | rev
        best = jnp.max(packed, axis=-1)
        ks.append(_NUM_VARS - 1 - (best & jnp.int32(63)))
    parts = []
    for p in range(_PAIRS):
        pidx = ks[2 * p] * _NUM_VARS + ks[2 * p + 1] + p * _NUM_VARS * _NUM_VARS
        parts.append(pidx.reshape(rows // _CHUNK, _CHUNK))
    idx_ref[...] = jnp.concatenate(parts, axis=0)


def _postproject_body(q_ref, wp_ref, bpost_ref, out_ref):
    r = q_ref[...].reshape(_PAIRS, _BLK, _PAIR_DIM)
    acc = None
    for p in range(_PAIRS):
        qp = r[p].astype(jnp.bfloat16)
        wp = wp_ref[p].astype(jnp.bfloat16)
        d = jnp.dot(qp, wp, preferred_element_type=jnp.float32)
        acc = d if acc is None else acc + d
    out_ref[...] = acc + bpost_ref[...]


def _make_sc_gather(rows, nch):
    mesh = plsc.VectorSubcoreMesh(core_axis_name="c", subcore_axis_name="s")

    @functools.partial(
        pl.kernel,
        mesh=mesh,
        out_type=jax.ShapeDtypeStruct((rows, _PAIR_DIM), jnp.float32),
        scratch_types=[
            pltpu.VMEM((nch, _CHUNK), jnp.int32),
            pltpu.VMEM((_CHUNK, _PAIR_DIM), jnp.float32),
            pltpu.VMEM((_CHUNK, _PAIR_DIM), jnp.float32),
            pltpu.SemaphoreType.DMA,
            pltpu.SemaphoreType.DMA,
        ],
    )
    def gather(idx_hbm, pt_hbm, out_hbm, idx_v, buf0, buf1, sem0, sem1):
        wid = lax.axis_index("s") * _NC + lax.axis_index("c")
        pltpu.sync_copy(idx_hbm.at[wid], idx_v)
        bufs = (buf0, buf1)
        sems = (sem0, sem1)
        base = wid * nch * _CHUNK
        cps = [None] * nch
        cps[0] = pltpu.async_copy(pt_hbm.at[idx_v.at[0]], bufs[0], sems[0])
        for j in range(nch):
            if j + 1 < nch:
                cps[j + 1] = pltpu.async_copy(
                    pt_hbm.at[idx_v.at[j + 1]], bufs[(j + 1) % 2], sems[(j + 1) % 2])
            cps[j].wait()
            pltpu.sync_copy(bufs[j % 2],
                            out_hbm.at[pl.ds(base + j * _CHUNK, _CHUNK)])

    return gather


def kernel(x, W_pre, b_pre, W_wp, b_wp, codebook, W_post, b_post):
    B, T, IN_DIM = x.shape
    OUT_DIM = W_post.shape[1]
    BT = B * T
    xf = x.reshape(BT, IN_DIM)
    bpre2 = b_pre.reshape(1, -1)
    bwp2 = b_wp.reshape(1, -1)
    bpost2 = b_post.reshape(1, -1)
    NN = _NUM_VARS * _NUM_VARS

    pair_table = pl.pallas_call(
        _pair_table_body,
        grid=(_PAIRS,),
        in_specs=[pl.BlockSpec((2 * _NUM_VARS, _VAR_DIM), lambda i: (i, 0))],
        out_specs=pl.BlockSpec((NN, _PAIR_DIM), lambda i: (i, 0)),
        out_shape=jax.ShapeDtypeStruct((_PAIRS * NN, _PAIR_DIM), jnp.float32),
    )(codebook)

    idx = pl.pallas_call(
        _logits_argmax_body,
        grid=(BT // _BLK,),
        in_specs=[
            pl.BlockSpec((_BLK, IN_DIM), lambda i: (i, 0)),
            pl.BlockSpec((IN_DIM, _PROJ_DIM), lambda i: (0, 0)),
            pl.BlockSpec((1, _PROJ_DIM), lambda i: (0, 0)),
            pl.BlockSpec((_PROJ_DIM, _GROUPS * _NUM_VARS), lambda i: (0, 0)),
            pl.BlockSpec((1, _GROUPS * _NUM_VARS), lambda i: (0, 0)),
        ],
        out_specs=pl.BlockSpec((_BLK * _PAIRS // _CHUNK, _CHUNK), lambda i: (i, 0)),
        out_shape=jax.ShapeDtypeStruct((BT * _PAIRS // _CHUNK, _CHUNK), jnp.int32),
    )(xf, W_pre, bpre2, W_wp, bwp2)

    rows = BT * _PAIRS
    nch = rows // (_NW * _CHUNK)
    q = _make_sc_gather(rows, nch)(idx.reshape(_NW, nch, _CHUNK), pair_table)

    out = pl.pallas_call(
        _postproject_body,
        grid=(BT // _BLK,),
        in_specs=[
            pl.BlockSpec((_BLK * _PAIRS, _PAIR_DIM), lambda i: (i, 0)),
            pl.BlockSpec((_PAIRS, _PAIR_DIM, OUT_DIM), lambda i: (0, 0, 0)),
            pl.BlockSpec((1, OUT_DIM), lambda i: (0, 0)),
        ],
        out_specs=pl.BlockSpec((_BLK, OUT_DIM), lambda i: (i, 0)),
        out_shape=jax.ShapeDtypeStruct((BT, OUT_DIM), jnp.float32),
    )(q, W_post.reshape(_PAIRS, _PAIR_DIM, OUT_DIM), bpost2)

    return out.reshape(B, T, OUT_DIM)


# speedup vs baseline: 2.1176x; 1.0096x over previous
import functools

import jax
import jax.numpy as jnp
from jax import lax
from jax.experimental import pallas as pl
from jax.experimental.pallas import tpu as pltpu
from jax.experimental.pallas import tpu_sc as plsc

_GROUPS = 8
_NUM_VARS = 64
_VAR_DIM = 64
_PROJ_DIM = 32
_PAIRS = 4
_PAIR_DIM = 2 * _VAR_DIM

_BLK = 512

_info = plsc.get_sparse_core_info()
_NC = _info.num_cores
_NS = _info.num_subcores
_NW = _NC * _NS
_CHUNK = 128


def _pair_table_body(cb_ref, pt_ref):
    cb0 = cb_ref[:_NUM_VARS, :]
    cb1 = cb_ref[_NUM_VARS:, :]
    a = jnp.broadcast_to(cb0[:, None, :], (_NUM_VARS, _NUM_VARS, _VAR_DIM))
    b = jnp.broadcast_to(cb1[None, :, :], (_NUM_VARS, _NUM_VARS, _VAR_DIM))
    a = a.reshape(_NUM_VARS * _NUM_VARS, _VAR_DIM)
    b = b.reshape(_NUM_VARS * _NUM_VARS, _VAR_DIM)
    pt_ref[...] = jnp.concatenate([a, b], axis=-1)


def _logits_argmax_body(x_ref, wpre_ref, bpre_ref, wwp_ref, bwp_ref, idx_ref):
    x = x_ref[...]
    h = jnp.dot(x, wpre_ref[...], preferred_element_type=jnp.float32)
    h = h + bpre_ref[...]
    logits = jnp.dot(h, wwp_ref[...], preferred_element_type=jnp.float32)
    logits = logits + bwp_ref[...]
    rows = x.shape[0]
    iot = jax.lax.broadcasted_iota(jnp.int32, (rows, _NUM_VARS), 1)
    rev = _NUM_VARS - 1 - iot
    minint = jnp.int32(-2147483648)
    ks = []
    for g in range(_GROUPS):
        sub = logits[:, g * _NUM_VARS:(g + 1) * _NUM_VARS]
        s = jax.lax.bitcast_convert_type(sub, jnp.int32)
        key = jnp.where(s >= 0, s, ~(s ^ minint))
        packed = (key & jnp.int32(~63)) | rev
        best = jnp.max(packed, axis=-1)
        ks.append(_NUM_VARS - 1 - (best & jnp.int32(63)))
    parts = []
    for p in range(_PAIRS):
        pidx = ks[2 * p] * _NUM_VARS + ks[2 * p + 1] + p * _NUM_VARS * _NUM_VARS
        parts.append(pidx.reshape(rows // _CHUNK, _CHUNK))
    idx_ref[...] = jnp.concatenate(parts, axis=0)


def _postproject_body(q_ref, wp_ref, bpost_ref, out_ref):
    r = q_ref[...].reshape(_PAIRS, _BLK, _PAIR_DIM)
    acc = None
    for p in range(_PAIRS):
        qp = r[p].astype(jnp.bfloat16)
        wp = wp_ref[p].astype(jnp.bfloat16)
        d = jnp.dot(qp, wp, preferred_element_type=jnp.float32)
        acc = d if acc is None else acc + d
    out_ref[...] = acc + bpost_ref[...]


def _postproject_alias_body(q_ref, wp_ref, bpost_ref, prev_ref, out_ref):
    del prev_ref
    _postproject_body(q_ref, wp_ref, bpost_ref, out_ref)


def _make_sc_gather(rows, nch):
    mesh = plsc.VectorSubcoreMesh(core_axis_name="c", subcore_axis_name="s")

    @functools.partial(
        pl.kernel,
        mesh=mesh,
        out_type=jax.ShapeDtypeStruct((rows, _PAIR_DIM), jnp.float32),
        scratch_types=[
            pltpu.VMEM((nch, _CHUNK), jnp.int32),
            pltpu.VMEM((_CHUNK, _PAIR_DIM), jnp.float32),
            pltpu.VMEM((_CHUNK, _PAIR_DIM), jnp.float32),
            pltpu.SemaphoreType.DMA,
            pltpu.SemaphoreType.DMA,
        ],
    )
    def gather(idx_hbm, pt_hbm, out_hbm, idx_v, buf0, buf1, sem0, sem1):
        wid = lax.axis_index("s") * _NC + lax.axis_index("c")
        pltpu.sync_copy(idx_hbm.at[wid], idx_v)
        bufs = (buf0, buf1)
        sems = (sem0, sem1)
        base = wid * nch * _CHUNK
        cps = [None] * nch
        cps[0] = pltpu.async_copy(pt_hbm.at[idx_v.at[0]], bufs[0], sems[0])
        for j in range(nch):
            if j + 1 < nch:
                cps[j + 1] = pltpu.async_copy(
                    pt_hbm.at[idx_v.at[j + 1]], bufs[(j + 1) % 2], sems[(j + 1) % 2])
            cps[j].wait()
            pltpu.sync_copy(bufs[j % 2],
                            out_hbm.at[pl.ds(base + j * _CHUNK, _CHUNK)])

    return gather


def kernel(x, W_pre, b_pre, W_wp, b_wp, codebook, W_post, b_post):
    B, T, IN_DIM = x.shape
    OUT_DIM = W_post.shape[1]
    BT = B * T
    xf = x.reshape(BT, IN_DIM)
    bpre2 = b_pre.reshape(1, -1)
    bwp2 = b_wp.reshape(1, -1)
    bpost2 = b_post.reshape(1, -1)
    NN = _NUM_VARS * _NUM_VARS

    pair_table = pl.pallas_call(
        _pair_table_body,
        grid=(_PAIRS,),
        in_specs=[pl.BlockSpec((2 * _NUM_VARS, _VAR_DIM), lambda i: (i, 0))],
        out_specs=pl.BlockSpec((NN, _PAIR_DIM), lambda i: (i, 0)),
        out_shape=jax.ShapeDtypeStruct((_PAIRS * NN, _PAIR_DIM), jnp.float32),
    )(codebook)

    THIRDS = 3
    BT3 = BT // THIRDS
    G3 = BT3 // _BLK
    rows3 = BT3 * _PAIRS
    nch = rows3 // (_NW * _CHUNK)
    sc_gather = _make_sc_gather(rows3, nch)
    wp3 = W_post.reshape(_PAIRS, _PAIR_DIM, OUT_DIM)

    def argmax_call(h):
        return pl.pallas_call(
            _logits_argmax_body,
            grid=(G3,),
            in_specs=[
                pl.BlockSpec((_BLK, IN_DIM), lambda i, h=h: (i + G3 * h, 0)),
                pl.BlockSpec((IN_DIM, _PROJ_DIM), lambda i: (0, 0)),
                pl.BlockSpec((1, _PROJ_DIM), lambda i: (0, 0)),
                pl.BlockSpec((_PROJ_DIM, _GROUPS * _NUM_VARS), lambda i: (0, 0)),
                pl.BlockSpec((1, _GROUPS * _NUM_VARS), lambda i: (0, 0)),
            ],
            out_specs=pl.BlockSpec((_BLK * _PAIRS // _CHUNK, _CHUNK),
                                   lambda i: (i, 0)),
            out_shape=jax.ShapeDtypeStruct((rows3 // _CHUNK, _CHUNK), jnp.int32),
        )(xf, W_pre, bpre2, W_wp, bwp2)

    def post_call(h, q, prev):
        qspec = pl.BlockSpec((_BLK * _PAIRS, _PAIR_DIM), lambda i: (i, 0))
        wspec = pl.BlockSpec((_PAIRS, _PAIR_DIM, OUT_DIM), lambda i: (0, 0, 0))
        bspec = pl.BlockSpec((1, OUT_DIM), lambda i: (0, 0))
        ospec = pl.BlockSpec((_BLK, OUT_DIM), lambda i, h=h: (i + G3 * h, 0))
        oshape = jax.ShapeDtypeStruct((BT, OUT_DIM), jnp.float32)
        if prev is None:
            return pl.pallas_call(
                _postproject_body, grid=(G3,),
                in_specs=[qspec, wspec, bspec],
                out_specs=ospec, out_shape=oshape,
            )(q, wp3, bpost2)
        return pl.pallas_call(
            _postproject_alias_body, grid=(G3,),
            in_specs=[qspec, wspec, bspec,
                      pl.BlockSpec(memory_space=pl.ANY)],
            out_specs=ospec, out_shape=oshape,
            input_output_aliases={3: 0},
        )(q, wp3, bpost2, prev)

    idxs = [argmax_call(h) for h in range(THIRDS)]
    qs = [sc_gather(idxs[h].reshape(_NW, nch, _CHUNK), pair_table)
          for h in range(THIRDS)]
    out = None
    for h in range(THIRDS):
        out = post_call(h, qs[h], out)

    return out.reshape(B, T, OUT_DIM)


# speedup vs baseline: 2.3473x; 1.1085x over previous
import functools

import jax
import jax.numpy as jnp
from jax import lax
from jax.experimental import pallas as pl
from jax.experimental.pallas import tpu as pltpu
from jax.experimental.pallas import tpu_sc as plsc

_GROUPS = 8
_NUM_VARS = 64
_VAR_DIM = 64
_PROJ_DIM = 32
_PAIRS = 4
_PAIR_DIM = 2 * _VAR_DIM

_BLK = 512

_info = plsc.get_sparse_core_info()
_NC = _info.num_cores
_NS = _info.num_subcores
_NW = _NC * _NS
_CHUNK = 128


def _pair_table_body(cb_ref, pt_ref):
    cb0 = cb_ref[:_NUM_VARS, :]
    cb1 = cb_ref[_NUM_VARS:, :]
    a = jnp.broadcast_to(cb0[:, None, :], (_NUM_VARS, _NUM_VARS, _VAR_DIM))
    b = jnp.broadcast_to(cb1[None, :, :], (_NUM_VARS, _NUM_VARS, _VAR_DIM))
    a = a.reshape(_NUM_VARS * _NUM_VARS, _VAR_DIM)
    b = b.reshape(_NUM_VARS * _NUM_VARS, _VAR_DIM)
    pt_ref[...] = jnp.concatenate([a, b], axis=-1)


def _logits_argmax_body(x_ref, wpre_ref, bpre_ref, wwp_ref, bwp_ref, idx_ref):
    x = x_ref[...]
    h = jnp.dot(x, wpre_ref[...], preferred_element_type=jnp.float32)
    h = h + bpre_ref[...]
    logits = jnp.dot(h, wwp_ref[...], preferred_element_type=jnp.float32)
    logits = logits + bwp_ref[...]
    rows = x.shape[0]
    iot = jax.lax.broadcasted_iota(jnp.int32, (rows, _GROUPS * _NUM_VARS), 1)
    rev = _NUM_VARS - 1 - (iot & jnp.int32(_NUM_VARS - 1))
    s = jax.lax.bitcast_convert_type(logits, jnp.int32)
    packed = jax.lax.bitcast_convert_type((s & jnp.int32(~63)) | rev,
                                          jnp.float32)
    ks = []
    for g in range(_GROUPS):
        best = jnp.max(packed[:, g * _NUM_VARS:(g + 1) * _NUM_VARS], axis=-1)
        bi = jax.lax.bitcast_convert_type(best, jnp.int32)
        ks.append(_NUM_VARS - 1 - (bi & jnp.int32(_NUM_VARS - 1)))
    parts = []
    for p in range(_PAIRS):
        pidx = ks[2 * p] * _NUM_VARS + ks[2 * p + 1] + p * _NUM_VARS * _NUM_VARS
        parts.append(pidx.reshape(rows // _CHUNK, _CHUNK))
    idx_ref[...] = jnp.concatenate(parts, axis=0)


def _postproject_body(q_ref, wp_ref, bpost_ref, out_ref):
    r = q_ref[...].reshape(_PAIRS, _BLK, _PAIR_DIM)
    acc = None
    for p in range(_PAIRS):
        qp = r[p].astype(jnp.bfloat16)
        wp = wp_ref[p * _PAIR_DIM:(p + 1) * _PAIR_DIM, :].astype(jnp.bfloat16)
        d = jnp.dot(qp, wp, preferred_element_type=jnp.float32)
        acc = d if acc is None else acc + d
    out_ref[...] = acc + bpost_ref[...]


def _postproject_alias_body(q_ref, wp_ref, bpost_ref, prev_ref, out_ref):
    del prev_ref
    _postproject_body(q_ref, wp_ref, bpost_ref, out_ref)


def _make_sc_gather(rows, nch):
    mesh = plsc.VectorSubcoreMesh(core_axis_name="c", subcore_axis_name="s")

    @functools.partial(
        pl.kernel,
        mesh=mesh,
        out_type=jax.ShapeDtypeStruct((rows, _PAIR_DIM), jnp.float32),
        scratch_types=[
            pltpu.VMEM((nch, _CHUNK), jnp.int32),
            pltpu.VMEM((_CHUNK, _PAIR_DIM), jnp.float32),
            pltpu.VMEM((_CHUNK, _PAIR_DIM), jnp.float32),
            pltpu.SemaphoreType.DMA,
            pltpu.SemaphoreType.DMA,
        ],
    )
    def gather(idx_hbm, pt_hbm, out_hbm, idx_v, buf0, buf1, sem0, sem1):
        wid = lax.axis_index("s") * _NC + lax.axis_index("c")
        pltpu.sync_copy(idx_hbm.at[wid], idx_v)
        bufs = (buf0, buf1)
        sems = (sem0, sem1)
        base = wid * nch * _CHUNK
        cps = [None] * nch
        cps[0] = pltpu.async_copy(pt_hbm.at[idx_v.at[0]], bufs[0], sems[0])
        for j in range(nch):
            if j + 1 < nch:
                cps[j + 1] = pltpu.async_copy(
                    pt_hbm.at[idx_v.at[j + 1]], bufs[(j + 1) % 2], sems[(j + 1) % 2])
            cps[j].wait()
            pltpu.sync_copy(bufs[j % 2],
                            out_hbm.at[pl.ds(base + j * _CHUNK, _CHUNK)])

    return gather


def kernel(x, W_pre, b_pre, W_wp, b_wp, codebook, W_post, b_post):
    B, T, IN_DIM = x.shape
    OUT_DIM = W_post.shape[1]
    BT = B * T
    xf = x.reshape(BT, IN_DIM)
    bpre2 = b_pre.reshape(1, -1)
    bwp2 = b_wp.reshape(1, -1)
    bpost2 = b_post.reshape(1, -1)
    NN = _NUM_VARS * _NUM_VARS

    pair_table = pl.pallas_call(
        _pair_table_body,
        grid=(_PAIRS,),
        in_specs=[pl.BlockSpec((2 * _NUM_VARS, _VAR_DIM), lambda i: (i, 0))],
        out_specs=pl.BlockSpec((NN, _PAIR_DIM), lambda i: (i, 0)),
        out_shape=jax.ShapeDtypeStruct((_PAIRS * NN, _PAIR_DIM), jnp.float32),
    )(codebook)

    THIRDS = 3
    BT3 = BT // THIRDS
    G3 = BT3 // _BLK
    rows3 = BT3 * _PAIRS
    nch = rows3 // (_NW * _CHUNK)
    sc_gather = _make_sc_gather(rows3, nch)

    def argmax_call(h):
        return pl.pallas_call(
            _logits_argmax_body,
            grid=(G3,),
            in_specs=[
                pl.BlockSpec((_BLK, IN_DIM), lambda i, h=h: (i + G3 * h, 0)),
                pl.BlockSpec((IN_DIM, _PROJ_DIM), lambda i: (0, 0)),
                pl.BlockSpec((1, _PROJ_DIM), lambda i: (0, 0)),
                pl.BlockSpec((_PROJ_DIM, _GROUPS * _NUM_VARS), lambda i: (0, 0)),
                pl.BlockSpec((1, _GROUPS * _NUM_VARS), lambda i: (0, 0)),
            ],
            out_specs=pl.BlockSpec((_BLK * _PAIRS // _CHUNK, _CHUNK),
                                   lambda i: (i, 0)),
            out_shape=jax.ShapeDtypeStruct((rows3 // _CHUNK, _CHUNK), jnp.int32),
        )(xf, W_pre, bpre2, W_wp, bwp2)

    def post_call(h, q, prev):
        qspec = pl.BlockSpec((_BLK * _PAIRS, _PAIR_DIM), lambda i: (i, 0))
        wspec = pl.BlockSpec((_PAIRS * _PAIR_DIM, OUT_DIM), lambda i: (0, 0))
        bspec = pl.BlockSpec((1, OUT_DIM), lambda i: (0, 0))
        ospec = pl.BlockSpec((_BLK, OUT_DIM), lambda i, h=h: (i + G3 * h, 0))
        oshape = jax.ShapeDtypeStruct((BT, OUT_DIM), jnp.float32)
        if prev is None:
            return pl.pallas_call(
                _postproject_body, grid=(G3,),
                in_specs=[qspec, wspec, bspec],
                out_specs=ospec, out_shape=oshape,
            )(q, W_post, bpost2)
        return pl.pallas_call(
            _postproject_alias_body, grid=(G3,),
            in_specs=[qspec, wspec, bspec,
                      pl.BlockSpec(memory_space=pl.ANY)],
            out_specs=ospec, out_shape=oshape,
            input_output_aliases={3: 0},
        )(q, W_post, bpost2, prev)

    idxs = [argmax_call(h) for h in range(THIRDS)]
    qs = [sc_gather(idxs[h].reshape(_NW, nch, _CHUNK), pair_table)
          for h in range(THIRDS)]
    out = None
    for h in range(THIRDS):
        out = post_call(h, qs[h], out)

    return out.reshape(B, T, OUT_DIM)


# speedup vs baseline: 2.6256x; 1.1185x over previous
import functools

import jax
import jax.numpy as jnp
from jax import lax
from jax.experimental import pallas as pl
from jax.experimental.pallas import tpu as pltpu
from jax.experimental.pallas import tpu_sc as plsc

_GROUPS = 8
_NUM_VARS = 64
_VAR_DIM = 64
_PROJ_DIM = 32
_PAIRS = 4
_PAIR_DIM = 2 * _VAR_DIM

_BLK = 1024

_info = plsc.get_sparse_core_info()
_NC = _info.num_cores
_NS = _info.num_subcores
_NW = _NC * _NS
_CHUNK = 128


def _pair_table_body(cb_ref, pt_ref):
    cb0 = cb_ref[:_NUM_VARS, :]
    cb1 = cb_ref[_NUM_VARS:, :]
    a = jnp.broadcast_to(cb0[:, None, :], (_NUM_VARS, _NUM_VARS, _VAR_DIM))
    b = jnp.broadcast_to(cb1[None, :, :], (_NUM_VARS, _NUM_VARS, _VAR_DIM))
    a = a.reshape(_NUM_VARS * _NUM_VARS, _VAR_DIM)
    b = b.reshape(_NUM_VARS * _NUM_VARS, _VAR_DIM)
    pt_ref[...] = jnp.concatenate([a, b], axis=-1)


def _logits_argmax_body(x_ref, wpre_ref, bpre_ref, wwp_ref, bwp_ref, idx_ref):
    x = x_ref[...]
    h = jnp.dot(x, wpre_ref[...], preferred_element_type=jnp.float32)
    h = h + bpre_ref[...]
    logits = jnp.dot(h, wwp_ref[...], preferred_element_type=jnp.float32)
    logits = logits + bwp_ref[...]
    rows = x.shape[0]
    iot = jax.lax.broadcasted_iota(jnp.int32, (rows, _GROUPS * _NUM_VARS), 1)
    rev = _NUM_VARS - 1 - (iot & jnp.int32(_NUM_VARS - 1))
    s = jax.lax.bitcast_convert_type(logits, jnp.int32)
    packed = jax.lax.bitcast_convert_type((s & jnp.int32(~63)) | rev,
                                          jnp.float32)
    ks = []
    for g in range(_GROUPS):
        best = jnp.max(packed[:, g * _NUM_VARS:(g + 1) * _NUM_VARS], axis=-1)
        bi = jax.lax.bitcast_convert_type(best, jnp.int32)
        ks.append(_NUM_VARS - 1 - (bi & jnp.int32(_NUM_VARS - 1)))
    parts = []
    for p in range(_PAIRS):
        pidx = ks[2 * p] * _NUM_VARS + ks[2 * p + 1] + p * _NUM_VARS * _NUM_VARS
        parts.append(pidx.reshape(rows // _CHUNK, _CHUNK))
    idx_ref[...] = jnp.concatenate(parts, axis=0)


def _postproject_body(q_ref, wp_ref, bpost_ref, out_ref):
    r = q_ref[...].reshape(_PAIRS, _BLK, _PAIR_DIM)
    acc = None
    for p in range(_PAIRS):
        qp = r[p].astype(jnp.bfloat16)
        wp = wp_ref[p * _PAIR_DIM:(p + 1) * _PAIR_DIM, :].astype(jnp.bfloat16)
        d = jnp.dot(qp, wp, preferred_element_type=jnp.float32)
        acc = d if acc is None else acc + d
    out_ref[...] = acc + bpost_ref[...]


def _postproject_alias_body(q_ref, wp_ref, bpost_ref, prev_ref, out_ref):
    del prev_ref
    _postproject_body(q_ref, wp_ref, bpost_ref, out_ref)


def _make_sc_gather(rows, nch):
    mesh = plsc.VectorSubcoreMesh(core_axis_name="c", subcore_axis_name="s")

    @functools.partial(
        pl.kernel,
        mesh=mesh,
        out_type=jax.ShapeDtypeStruct((rows, _PAIR_DIM), jnp.float32),
        scratch_types=[
            pltpu.VMEM((nch, _CHUNK), jnp.int32),
            pltpu.VMEM((_CHUNK, _PAIR_DIM), jnp.float32),
            pltpu.VMEM((_CHUNK, _PAIR_DIM), jnp.float32),
            pltpu.SemaphoreType.DMA,
            pltpu.SemaphoreType.DMA,
        ],
    )
    def gather(idx_hbm, pt_hbm, out_hbm, idx_v, buf0, buf1, sem0, sem1):
        wid = lax.axis_index("s") * _NC + lax.axis_index("c")
        pltpu.sync_copy(idx_hbm.at[wid], idx_v)
        bufs = (buf0, buf1)
        sems = (sem0, sem1)
        base = wid * nch * _CHUNK
        cps = [None] * nch
        cps[0] = pltpu.async_copy(pt_hbm.at[idx_v.at[0]], bufs[0], sems[0])
        for j in range(nch):
            if j + 1 < nch:
                cps[j + 1] = pltpu.async_copy(
                    pt_hbm.at[idx_v.at[j + 1]], bufs[(j + 1) % 2], sems[(j + 1) % 2])
            cps[j].wait()
            pltpu.sync_copy(bufs[j % 2],
                            out_hbm.at[pl.ds(base + j * _CHUNK, _CHUNK)])

    return gather


def kernel(x, W_pre, b_pre, W_wp, b_wp, codebook, W_post, b_post):
    B, T, IN_DIM = x.shape
    OUT_DIM = W_post.shape[1]
    BT = B * T
    xf = x.reshape(BT, IN_DIM)
    bpre2 = b_pre.reshape(1, -1)
    bwp2 = b_wp.reshape(1, -1)
    bpost2 = b_post.reshape(1, -1)
    NN = _NUM_VARS * _NUM_VARS

    pair_table = pl.pallas_call(
        _pair_table_body,
        grid=(_PAIRS,),
        in_specs=[pl.BlockSpec((2 * _NUM_VARS, _VAR_DIM), lambda i: (i, 0))],
        out_specs=pl.BlockSpec((NN, _PAIR_DIM), lambda i: (i, 0)),
        out_shape=jax.ShapeDtypeStruct((_PAIRS * NN, _PAIR_DIM), jnp.float32),
    )(codebook)

    THIRDS = 3
    BT3 = BT // THIRDS
    G3 = BT3 // _BLK
    rows3 = BT3 * _PAIRS
    nch = rows3 // (_NW * _CHUNK)
    sc_gather = _make_sc_gather(rows3, nch)

    def argmax_call(h):
        return pl.pallas_call(
            _logits_argmax_body,
            grid=(G3,),
            in_specs=[
                pl.BlockSpec((_BLK, IN_DIM), lambda i, h=h: (i + G3 * h, 0)),
                pl.BlockSpec((IN_DIM, _PROJ_DIM), lambda i: (0, 0)),
                pl.BlockSpec((1, _PROJ_DIM), lambda i: (0, 0)),
                pl.BlockSpec((_PROJ_DIM, _GROUPS * _NUM_VARS), lambda i: (0, 0)),
                pl.BlockSpec((1, _GROUPS * _NUM_VARS), lambda i: (0, 0)),
            ],
            out_specs=pl.BlockSpec((_BLK * _PAIRS // _CHUNK, _CHUNK),
                                   lambda i: (i, 0)),
            out_shape=jax.ShapeDtypeStruct((rows3 // _CHUNK, _CHUNK), jnp.int32),
        )(xf, W_pre, bpre2, W_wp, bwp2)

    def post_call(h, q, prev):
        qspec = pl.BlockSpec((_BLK * _PAIRS, _PAIR_DIM), lambda i: (i, 0))
        wspec = pl.BlockSpec((_PAIRS * _PAIR_DIM, OUT_DIM), lambda i: (0, 0))
        bspec = pl.BlockSpec((1, OUT_DIM), lambda i: (0, 0))
        ospec = pl.BlockSpec((_BLK, OUT_DIM), lambda i, h=h: (i + G3 * h, 0))
        oshape = jax.ShapeDtypeStruct((BT, OUT_DIM), jnp.float32)
        if prev is None:
            return pl.pallas_call(
                _postproject_body, grid=(G3,),
                in_specs=[qspec, wspec, bspec],
                out_specs=ospec, out_shape=oshape,
            )(q, W_post, bpost2)
        return pl.pallas_call(
            _postproject_alias_body, grid=(G3,),
            in_specs=[qspec, wspec, bspec,
                      pl.BlockSpec(memory_space=pl.ANY)],
            out_specs=ospec, out_shape=oshape,
            input_output_aliases={3: 0},
        )(q, W_post, bpost2, prev)

    idxs = [argmax_call(h) for h in range(THIRDS)]
    qs = [sc_gather(idxs[h].reshape(_NW, nch, _CHUNK), pair_table)
          for h in range(THIRDS)]
    out = None
    for h in range(THIRDS):
        out = post_call(h, qs[h], out)

    return out.reshape(B, T, OUT_DIM)


# speedup vs baseline: 2.6377x; 1.0046x over previous
import functools

import jax
import jax.numpy as jnp
from jax import lax
from jax.experimental import pallas as pl
from jax.experimental.pallas import tpu as pltpu
from jax.experimental.pallas import tpu_sc as plsc

_GROUPS = 8
_NUM_VARS = 64
_VAR_DIM = 64
_PROJ_DIM = 32
_PAIRS = 4
_PAIR_DIM = 2 * _VAR_DIM

_BLK = 1536

_info = plsc.get_sparse_core_info()
_NC = _info.num_cores
_NS = _info.num_subcores
_NW = _NC * _NS
_CHUNK = 128


def _pair_table_body(cb_ref, pt_ref):
    cb0 = cb_ref[:_NUM_VARS, :]
    cb1 = cb_ref[_NUM_VARS:, :]
    a = jnp.broadcast_to(cb0[:, None, :], (_NUM_VARS, _NUM_VARS, _VAR_DIM))
    b = jnp.broadcast_to(cb1[None, :, :], (_NUM_VARS, _NUM_VARS, _VAR_DIM))
    a = a.reshape(_NUM_VARS * _NUM_VARS, _VAR_DIM)
    b = b.reshape(_NUM_VARS * _NUM_VARS, _VAR_DIM)
    pt_ref[...] = jnp.concatenate([a, b], axis=-1)


def _logits_argmax_body(x_ref, wpre_ref, bpre_ref, wwp_ref, bwp_ref, idx_ref):
    x = x_ref[...]
    h = jnp.dot(x, wpre_ref[...], preferred_element_type=jnp.float32)
    h = h + bpre_ref[...]
    logits = jnp.dot(h, wwp_ref[...], preferred_element_type=jnp.float32)
    logits = logits + bwp_ref[...]
    rows = x.shape[0]
    iot = jax.lax.broadcasted_iota(jnp.int32, (rows, _GROUPS * _NUM_VARS), 1)
    rev = _NUM_VARS - 1 - (iot & jnp.int32(_NUM_VARS - 1))
    s = jax.lax.bitcast_convert_type(logits, jnp.int32)
    packed = jax.lax.bitcast_convert_type((s & jnp.int32(~63)) | rev,
                                          jnp.float32)
    ks = []
    for g in range(_GROUPS):
        best = jnp.max(packed[:, g * _NUM_VARS:(g + 1) * _NUM_VARS], axis=-1)
        bi = jax.lax.bitcast_convert_type(best, jnp.int32)
        ks.append(_NUM_VARS - 1 - (bi & jnp.int32(_NUM_VARS - 1)))
    parts = []
    for p in range(_PAIRS):
        pidx = ks[2 * p] * _NUM_VARS + ks[2 * p + 1] + p * _NUM_VARS * _NUM_VARS
        parts.append(pidx.reshape(rows // _CHUNK, _CHUNK))
    idx_ref[...] = jnp.concatenate(parts, axis=0)


def _postproject_body(q_ref, wp_ref, bpost_ref, out_ref):
    r = q_ref[...].reshape(_PAIRS, _BLK, _PAIR_DIM)
    acc = None
    for p in range(_PAIRS):
        qp = r[p].astype(jnp.bfloat16)
        wp = wp_ref[p * _PAIR_DIM:(p + 1) * _PAIR_DIM, :].astype(jnp.bfloat16)
        d = jnp.dot(qp, wp, preferred_element_type=jnp.float32)
        acc = d if acc is None else acc + d
    out_ref[...] = acc + bpost_ref[...]


def _postproject_alias_body(q_ref, wp_ref, bpost_ref, prev_ref, out_ref):
    del prev_ref
    _postproject_body(q_ref, wp_ref, bpost_ref, out_ref)


def _make_sc_gather(rows, nch):
    mesh = plsc.VectorSubcoreMesh(core_axis_name="c", subcore_axis_name="s")

    @functools.partial(
        pl.kernel,
        mesh=mesh,
        out_type=jax.ShapeDtypeStruct((rows, _PAIR_DIM), jnp.float32),
        scratch_types=[
            pltpu.VMEM((nch, _CHUNK), jnp.int32),
            pltpu.VMEM((_CHUNK, _PAIR_DIM), jnp.float32),
            pltpu.VMEM((_CHUNK, _PAIR_DIM), jnp.float32),
            pltpu.SemaphoreType.DMA,
            pltpu.SemaphoreType.DMA,
        ],
    )
    def gather(idx_hbm, pt_hbm, out_hbm, idx_v, buf0, buf1, sem0, sem1):
        wid = lax.axis_index("s") * _NC + lax.axis_index("c")
        pltpu.sync_copy(idx_hbm.at[wid], idx_v)
        bufs = (buf0, buf1)
        sems = (sem0, sem1)
        base = wid * nch * _CHUNK
        cps = [None] * nch
        cps[0] = pltpu.async_copy(pt_hbm.at[idx_v.at[0]], bufs[0], sems[0])
        for j in range(nch):
            if j + 1 < nch:
                cps[j + 1] = pltpu.async_copy(
                    pt_hbm.at[idx_v.at[j + 1]], bufs[(j + 1) % 2], sems[(j + 1) % 2])
            cps[j].wait()
            pltpu.sync_copy(bufs[j % 2],
                            out_hbm.at[pl.ds(base + j * _CHUNK, _CHUNK)])

    return gather


def kernel(x, W_pre, b_pre, W_wp, b_wp, codebook, W_post, b_post):
    B, T, IN_DIM = x.shape
    OUT_DIM = W_post.shape[1]
    BT = B * T
    xf = x.reshape(BT, IN_DIM)
    bpre2 = b_pre.reshape(1, -1)
    bwp2 = b_wp.reshape(1, -1)
    bpost2 = b_post.reshape(1, -1)
    NN = _NUM_VARS * _NUM_VARS

    pair_table = pl.pallas_call(
        _pair_table_body,
        grid=(_PAIRS,),
        in_specs=[pl.BlockSpec((2 * _NUM_VARS, _VAR_DIM), lambda i: (i, 0))],
        out_specs=pl.BlockSpec((NN, _PAIR_DIM), lambda i: (i, 0)),
        out_shape=jax.ShapeDtypeStruct((_PAIRS * NN, _PAIR_DIM), jnp.float32),
    )(codebook)

    THIRDS = 3
    BT3 = BT // THIRDS
    G3 = BT3 // _BLK
    rows3 = BT3 * _PAIRS
    nch = rows3 // (_NW * _CHUNK)
    sc_gather = _make_sc_gather(rows3, nch)

    def argmax_call(h):
        return pl.pallas_call(
            _logits_argmax_body,
            grid=(G3,),
            in_specs=[
                pl.BlockSpec((_BLK, IN_DIM), lambda i, h=h: (i + G3 * h, 0)),
                pl.BlockSpec((IN_DIM, _PROJ_DIM), lambda i: (0, 0)),
                pl.BlockSpec((1, _PROJ_DIM), lambda i: (0, 0)),
                pl.BlockSpec((_PROJ_DIM, _GROUPS * _NUM_VARS), lambda i: (0, 0)),
                pl.BlockSpec((1, _GROUPS * _NUM_VARS), lambda i: (0, 0)),
            ],
            out_specs=pl.BlockSpec((_BLK * _PAIRS // _CHUNK, _CHUNK),
                                   lambda i: (i, 0)),
            out_shape=jax.ShapeDtypeStruct((rows3 // _CHUNK, _CHUNK), jnp.int32),
        )(xf, W_pre, bpre2, W_wp, bwp2)

    def post_call(h, q, prev):
        qspec = pl.BlockSpec((_BLK * _PAIRS, _PAIR_DIM), lambda i: (i, 0))
        wspec = pl.BlockSpec((_PAIRS * _PAIR_DIM, OUT_DIM), lambda i: (0, 0))
        bspec = pl.BlockSpec((1, OUT_DIM), lambda i: (0, 0))
        ospec = pl.BlockSpec((_BLK, OUT_DIM), lambda i, h=h: (i + G3 * h, 0))
        oshape = jax.ShapeDtypeStruct((BT, OUT_DIM), jnp.float32)
        if prev is None:
            return pl.pallas_call(
                _postproject_body, grid=(G3,),
                in_specs=[qspec, wspec, bspec],
                out_specs=ospec, out_shape=oshape,
            )(q, W_post, bpost2)
        return pl.pallas_call(
            _postproject_alias_body, grid=(G3,),
            in_specs=[qspec, wspec, bspec,
                      pl.BlockSpec(memory_space=pl.ANY)],
            out_specs=ospec, out_shape=oshape,
            input_output_aliases={3: 0},
        )(q, W_post, bpost2, prev)

    idxs = [argmax_call(h) for h in range(THIRDS)]
    qs = [sc_gather(idxs[h].reshape(_NW, nch, _CHUNK), pair_table)
          for h in range(THIRDS)]
    out = None
    for h in range(THIRDS):
        out = post_call(h, qs[h], out)

    return out.reshape(B, T, OUT_DIM)
